# baseline (speedup 1.0000x reference)
import functools

import jax
import jax.numpy as jnp
from jax import lax
from jax.experimental import pallas as pl
from jax.experimental.pallas import tpu as pltpu
from jax.experimental.pallas import tpu_sc as plsc

NSAMPLE = 32
LEAKY_RATE = 0.1
NB = 512
GRP = 8
NW = 32
INTMAX = 2**31 - 1
FMAX = 3.4028235e38


def _knn_kernel(xyz_ref, pts_ref, w_ref, idx_ref, g_ref, *, n):
    i = pl.program_id(1)
    xcc = xyz_ref[0, :, pl.ds(i * NB, NB)]
    xc = jnp.transpose(xcc)

    feat = jnp.concatenate([xcc, pts_ref[0]], axis=0)
    gcols = jnp.dot(w_ref[...], feat, preferred_element_type=jnp.float32)
    g_ref[0] = jnp.transpose(gcols)

    sqc = jnp.sum(xc * xc, axis=1, keepdims=True)

    w = n // 8
    ks = []
    for j in range(8):
        xfj = xyz_ref[0, :, pl.ds(j * w, w)]
        sqfj = jnp.sum(xfj * xfj, axis=0, keepdims=True)
        dj = sqc + sqfj - 2.0 * jnp.dot(xc, xfj,
                                        preferred_element_type=jnp.float32)
        d0j = jnp.maximum(dj, 0.0)
        iotaj = lax.broadcasted_iota(jnp.int32, (NB, w), 1) + jnp.int32(j * w)
        ks.append(lax.bitcast_convert_type(
            ((lax.bitcast_convert_type(d0j, jnp.int32) & jnp.int32(~4095))
             | iotaj) + jnp.int32(0x10000000),
            jnp.float32))

    mn, mx = jnp.minimum, jnp.maximum

    def merge22(p, q):
        c1 = mn(p[0], q[0])
        x = mx(p[0], q[0])
        y = mn(p[1], q[1])
        return c1, mn(x, y), mx(x, y)

    def merge33(p, q):
        c1 = mn(p[0], q[0])
        t1 = mx(p[0], q[0])
        u = mn(p[1], q[1])
        c2 = mn(t1, u)
        w1 = mx(t1, u)
        v = mx(p[1], q[1])
        w2 = mn(v, mn(p[2], q[2]))
        return c1, c2, mn(w1, w2)

    s2 = [(mn(ks[2 * j], ks[2 * j + 1]), mx(ks[2 * j], ks[2 * j + 1]))
          for j in range(4)]
    pool = jnp.concatenate(
        merge33(merge22(s2[0], s2[1]), merge22(s2[2], s2[3])), axis=1)

    w2 = pool.shape[1] // 6
    ps = [pool[:, j * w2:(j + 1) * w2] for j in range(6)]
    t2 = [(mn(ps[2 * j], ps[2 * j + 1]), mx(ps[2 * j], ps[2 * j + 1]))
          for j in range(3)]
    s3b = (t2[2][0], t2[2][1], jnp.full((NB, w2), FMAX, jnp.float32))
    pool = jnp.concatenate(merge33(merge22(t2[0], t2[1]), s3b), axis=1)

    w3 = pool.shape[1] // 6
    ps = [pool[:, j * w3:(j + 1) * w3] for j in range(6)]
    t3 = [(mn(ps[2 * j], ps[2 * j + 1]), mx(ps[2 * j], ps[2 * j + 1]))
          for j in range(3)]
    (a1, a2), (b1, b2) = t3[0], t3[1]
    t = mx(a1, b1)
    u = mn(a2, b2)
    s4 = (mn(a1, b1), mn(t, u), mx(t, u), mx(a2, b2))
    (a1, a2, a3, a4), (b1, b2) = s4, t3[2]
    pool = jnp.concatenate([
        mn(a1, b1),
        mn(mx(a1, b1), mn(a2, b2)),
        mn(a3, mn(mx(a2, b1), mx(a1, b2))),
        mn(a4, mn(mx(a3, b1), mx(a2, b2))),
    ], axis=1)

    ms = []
    for it in range(NSAMPLE):
        m = jnp.min(pool, axis=1, keepdims=True)
        if it + 1 < NSAMPLE:
            pool = jnp.where(pool == m, jnp.float32(FMAX), pool)
        ms.append(m)
    idx = lax.bitcast_convert_type(
        jnp.concatenate(ms, axis=1), jnp.int32) & jnp.int32(4095)
    idx_ref[0] = idx + pl.program_id(0) * n


def _sc_gmax(g_hbm, idx_hbm, out_hbm, idx_v, rows0, rows1, ostage, sem0, sem1,
             *, cpw, ngrp):
    k = NSAMPLE
    wid = lax.axis_index("s") * 2 + lax.axis_index("c")
    base_c = wid * cpw
    pltpu.sync_copy(idx_hbm.at[pl.ds(base_c * k, cpw * k)], idx_v)

    def fire(g, rows, sem):
        pltpu.async_copy(
            g_hbm.at[idx_v.at[pl.ds(g * (GRP * k), GRP * k)]], rows, sem)

    def wait(g, rows, sem):
        pltpu.make_async_copy(
            g_hbm.at[idx_v.at[pl.ds(g * (GRP * k), GRP * k)]], rows, sem).wait()

    def process(g, rows):
        for c8 in range(GRP):
            def rbody(j, accs, c8=c8):
                r = c8 * k + j
                return tuple(
                    jnp.maximum(accs[c], rows[r, pl.ds(c * 16, 16)])
                    for c in range(8))
            accs = tuple(rows[c8 * k, pl.ds(c * 16, 16)] for c in range(8))
            accs = lax.fori_loop(1, k, rbody, accs)
            for c in range(8):
                ostage[c8, pl.ds(c * 16, 16)] = accs[c]
        pltpu.sync_copy(ostage, out_hbm.at[pl.ds(base_c + g * GRP, GRP)])

    fire(0, rows0, sem0)

    def body(i2, carry):
        g0 = 2 * i2
        fire(g0 + 1, rows1, sem1)
        wait(g0, rows0, sem0)
        process(g0, rows0)

        @pl.when(g0 + 2 < ngrp)
        def _():
            fire(g0 + 2, rows0, sem0)

        wait(g0 + 1, rows1, sem1)
        process(g0 + 1, rows1)
        return carry

    lax.fori_loop(0, ngrp // 2, body, 0)


def _epi_kernel(maxf_ref, xyz_ref, w_ref, b_ref, out_ref):
    mft = jnp.transpose(maxf_ref[0])
    c = jnp.dot(w_ref[:, 0:3], xyz_ref[0], preferred_element_type=jnp.float32)
    res = mft - c + b_ref[...]
    res = jnp.where(res > 0, res, LEAKY_RATE * res)
    out_ref[0] = res


def kernel(xyz, points, W, b):
    B, C, N = xyz.shape
    D = points.shape[1]
    OC = W.shape[0]
    BN = B * N
    f32 = jnp.float32

    b_col = b[:, None]

    idx, g_rows = pl.pallas_call(
        functools.partial(_knn_kernel, n=N),
        grid=(B, N // NB),
        in_specs=[
            pl.BlockSpec((1, C, N), lambda bi, i: (bi, 0, 0)),
            pl.BlockSpec((1, D, NB), lambda bi, i: (bi, 0, i)),
            pl.BlockSpec((OC, C + D), lambda bi, i: (0, 0)),
        ],
        out_specs=[
            pl.BlockSpec((1, NB, NSAMPLE), lambda bi, i: (bi, i, 0)),
            pl.BlockSpec((1, NB, OC), lambda bi, i: (bi, i, 0)),
        ],
        out_shape=[
            jax.ShapeDtypeStruct((B, N, NSAMPLE), jnp.int32),
            jax.ShapeDtypeStruct((B, N, OC), f32),
        ],
    )(xyz, points, W)

    g2 = jnp.reshape(g_rows, (BN, OC))
    idxflat = jnp.reshape(idx, (BN * NSAMPLE,))

    cpw = BN // NW
    ngrp = cpw // GRP
    mesh = plsc.VectorSubcoreMesh(core_axis_name="c", subcore_axis_name="s")
    maxf = pl.kernel(
        functools.partial(_sc_gmax, cpw=cpw, ngrp=ngrp),
        out_type=jax.ShapeDtypeStruct((BN, OC), f32),
        mesh=mesh,
        scratch_types=[
            pltpu.VMEM((cpw * NSAMPLE,), jnp.int32),
            pltpu.VMEM((GRP * NSAMPLE, OC), f32),
            pltpu.VMEM((GRP * NSAMPLE, OC), f32),
            pltpu.VMEM((GRP, OC), f32),
            pltpu.SemaphoreType.DMA,
            pltpu.SemaphoreType.DMA,
        ],
    )(g2, idxflat)

    out = pl.pallas_call(
        _epi_kernel,
        grid=(B, N // 512),
        in_specs=[
            pl.BlockSpec((1, 512, OC), lambda bi, i: (bi, i, 0)),
            pl.BlockSpec((1, C, 512), lambda bi, i: (bi, 0, i)),
            pl.BlockSpec((OC, C + D), lambda bi, i: (0, 0)),
            pl.BlockSpec((OC, 1), lambda bi, i: (0, 0)),
        ],
        out_specs=pl.BlockSpec((1, OC, 512), lambda bi, i: (bi, 0, i)),
        out_shape=jax.ShapeDtypeStruct((B, OC, N), f32),
    )(jnp.reshape(maxf, (B, N, OC)), xyz, W, b_col)
    return out


# speedup vs baseline: 1.0094x; 1.0094x over previous
import functools

import jax
import jax.numpy as jnp
from jax import lax
from jax.experimental import pallas as pl
from jax.experimental.pallas import tpu as pltpu
from jax.experimental.pallas import tpu_sc as plsc

NSAMPLE = 32
LEAKY_RATE = 0.1
NB = 512
GRP = 8
NW = 32
INTMAX = 2**31 - 1
FMAX = 3.4028235e38


def _knn_kernel(xyz8_ref, pts_ref, w_ref, idx_ref, g_ref, *, n):
    i = pl.program_id(1)
    xcc = xyz8_ref[0, :, pl.ds(i * NB, NB)]
    xc = jnp.transpose(xcc)

    feat = jnp.concatenate([xcc[:3], pts_ref[0]], axis=0)
    gcols = jnp.dot(w_ref[...], feat, preferred_element_type=jnp.float32)
    g_ref[0] = jnp.transpose(gcols)

    sqc = jnp.sum(xc * xc, axis=1, keepdims=True)

    w = n // 8
    ks = []
    for j in range(8):
        xfj = xyz8_ref[0, :, pl.ds(j * w, w)]
        sqfj = jnp.sum(xfj * xfj, axis=0, keepdims=True)
        dj = sqc + sqfj - 2.0 * jnp.dot(xc, xfj,
                                        preferred_element_type=jnp.float32)
        d0j = jnp.maximum(dj, 0.0)
        iotaj = lax.broadcasted_iota(jnp.int32, (NB, w), 1) + jnp.int32(j * w)
        ks.append(lax.bitcast_convert_type(
            ((lax.bitcast_convert_type(d0j, jnp.int32) & jnp.int32(~4095))
             | iotaj) + jnp.int32(0x10000000),
            jnp.float32))

    mn, mx = jnp.minimum, jnp.maximum

    def merge22(p, q):
        c1 = mn(p[0], q[0])
        x = mx(p[0], q[0])
        y = mn(p[1], q[1])
        return c1, mn(x, y), mx(x, y)

    def merge33(p, q):
        c1 = mn(p[0], q[0])
        t1 = mx(p[0], q[0])
        u = mn(p[1], q[1])
        c2 = mn(t1, u)
        w1 = mx(t1, u)
        v = mx(p[1], q[1])
        w2 = mn(v, mn(p[2], q[2]))
        return c1, c2, mn(w1, w2)

    s2 = [(mn(ks[2 * j], ks[2 * j + 1]), mx(ks[2 * j], ks[2 * j + 1]))
          for j in range(4)]
    pool = jnp.concatenate(
        merge33(merge22(s2[0], s2[1]), merge22(s2[2], s2[3])), axis=1)

    w2 = pool.shape[1] // 6
    ps = [pool[:, j * w2:(j + 1) * w2] for j in range(6)]
    t2 = [(mn(ps[2 * j], ps[2 * j + 1]), mx(ps[2 * j], ps[2 * j + 1]))
          for j in range(3)]
    s3b = (t2[2][0], t2[2][1], jnp.full((NB, w2), FMAX, jnp.float32))
    pool = jnp.concatenate(merge33(merge22(t2[0], t2[1]), s3b), axis=1)

    w3 = pool.shape[1] // 6
    ps = [pool[:, j * w3:(j + 1) * w3] for j in range(6)]
    t3 = [(mn(ps[2 * j], ps[2 * j + 1]), mx(ps[2 * j], ps[2 * j + 1]))
          for j in range(3)]
    (a1, a2), (b1, b2) = t3[0], t3[1]
    t = mx(a1, b1)
    u = mn(a2, b2)
    s4 = (mn(a1, b1), mn(t, u), mx(t, u), mx(a2, b2))
    (a1, a2, a3, a4), (b1, b2) = s4, t3[2]
    pool = jnp.concatenate([
        mn(a1, b1),
        mn(mx(a1, b1), mn(a2, b2)),
        mn(a3, mn(mx(a2, b1), mx(a1, b2))),
        mn(a4, mn(mx(a3, b1), mx(a2, b2))),
    ], axis=1)

    ms = []
    for it in range(NSAMPLE):
        m = jnp.min(pool, axis=1, keepdims=True)
        if it + 1 < NSAMPLE:
            pool = jnp.where(pool == m, jnp.float32(FMAX), pool)
        ms.append(m)
    idx = lax.bitcast_convert_type(
        jnp.concatenate(ms, axis=1), jnp.int32) & jnp.int32(4095)
    idx_ref[0] = idx + pl.program_id(0) * n


def _sc_gmax(g_hbm, idx_hbm, out_hbm, idx_v, rows0, rows1, ostage, sem0, sem1,
             *, cpw, ngrp):
    k = NSAMPLE
    wid = lax.axis_index("s") * 2 + lax.axis_index("c")
    base_c = wid * cpw
    pltpu.sync_copy(idx_hbm.at[pl.ds(base_c * k, cpw * k)], idx_v)

    def fire(g, rows, sem):
        pltpu.async_copy(
            g_hbm.at[idx_v.at[pl.ds(g * (GRP * k), GRP * k)]], rows, sem)

    def wait(g, rows, sem):
        pltpu.make_async_copy(
            g_hbm.at[idx_v.at[pl.ds(g * (GRP * k), GRP * k)]], rows, sem).wait()

    def process(g, rows):
        for c8 in range(GRP):
            def rbody(j, accs, c8=c8):
                r = c8 * k + j
                return tuple(
                    jnp.maximum(accs[c], rows[r, pl.ds(c * 16, 16)])
                    for c in range(8))
            accs = tuple(rows[c8 * k, pl.ds(c * 16, 16)] for c in range(8))
            accs = lax.fori_loop(1, k, rbody, accs)
            for c in range(8):
                ostage[c8, pl.ds(c * 16, 16)] = accs[c]
        pltpu.sync_copy(ostage, out_hbm.at[pl.ds(base_c + g * GRP, GRP)])

    fire(0, rows0, sem0)

    def body(i2, carry):
        g0 = 2 * i2
        fire(g0 + 1, rows1, sem1)
        wait(g0, rows0, sem0)
        process(g0, rows0)

        @pl.when(g0 + 2 < ngrp)
        def _():
            fire(g0 + 2, rows0, sem0)

        wait(g0 + 1, rows1, sem1)
        process(g0 + 1, rows1)
        return carry

    lax.fori_loop(0, ngrp // 2, body, 0)


def _epi_kernel(maxf_ref, xyz8_ref, w38_ref, b_ref, out_ref):
    mft = jnp.transpose(maxf_ref[0])
    c = jnp.dot(w38_ref[...], xyz8_ref[0], preferred_element_type=jnp.float32)
    res = mft - c + b_ref[...]
    res = jnp.where(res > 0, res, LEAKY_RATE * res)
    out_ref[0] = res


def kernel(xyz, points, W, b):
    B, C, N = xyz.shape
    D = points.shape[1]
    OC = W.shape[0]
    BN = B * N
    f32 = jnp.float32

    xyz8 = jnp.concatenate([xyz, jnp.zeros((B, 5, N), f32)], axis=1)
    W38 = jnp.concatenate([W[:, :3], jnp.zeros((OC, 5), f32)], axis=1)
    b_col = b[:, None]

    idx, g_rows = pl.pallas_call(
        functools.partial(_knn_kernel, n=N),
        grid=(B, N // NB),
        in_specs=[
            pl.BlockSpec((1, 8, N), lambda bi, i: (bi, 0, 0)),
            pl.BlockSpec((1, D, NB), lambda bi, i: (bi, 0, i)),
            pl.BlockSpec((OC, C + D), lambda bi, i: (0, 0)),
        ],
        out_specs=[
            pl.BlockSpec((1, NB, NSAMPLE), lambda bi, i: (bi, i, 0)),
            pl.BlockSpec((1, NB, OC), lambda bi, i: (bi, i, 0)),
        ],
        out_shape=[
            jax.ShapeDtypeStruct((B, N, NSAMPLE), jnp.int32),
            jax.ShapeDtypeStruct((B, N, OC), f32),
        ],
    )(xyz8, points, W)

    g2 = jnp.reshape(g_rows, (BN, OC))
    idxflat = jnp.reshape(idx, (BN * NSAMPLE,))

    cpw = BN // NW
    ngrp = cpw // GRP
    mesh = plsc.VectorSubcoreMesh(core_axis_name="c", subcore_axis_name="s")
    maxf = pl.kernel(
        functools.partial(_sc_gmax, cpw=cpw, ngrp=ngrp),
        out_type=jax.ShapeDtypeStruct((BN, OC), f32),
        mesh=mesh,
        scratch_types=[
            pltpu.VMEM((cpw * NSAMPLE,), jnp.int32),
            pltpu.VMEM((GRP * NSAMPLE, OC), f32),
            pltpu.VMEM((GRP * NSAMPLE, OC), f32),
            pltpu.VMEM((GRP, OC), f32),
            pltpu.SemaphoreType.DMA,
            pltpu.SemaphoreType.DMA,
        ],
    )(g2, idxflat)

    out = pl.pallas_call(
        _epi_kernel,
        grid=(B, N // 512),
        in_specs=[
            pl.BlockSpec((1, 512, OC), lambda bi, i: (bi, i, 0)),
            pl.BlockSpec((1, 8, 512), lambda bi, i: (bi, 0, i)),
            pl.BlockSpec((OC, 8), lambda bi, i: (0, 0)),
            pl.BlockSpec((OC, 1), lambda bi, i: (0, 0)),
        ],
        out_specs=pl.BlockSpec((1, OC, 512), lambda bi, i: (bi, 0, i)),
        out_shape=jax.ShapeDtypeStruct((B, OC, N), f32),
    )(jnp.reshape(maxf, (B, N, OC)), xyz8, W38, b_col)
    return out
